# baseline (speedup 1.0000x reference)
import jax
import jax.numpy as jnp
from jax import lax
from jax.experimental import pallas as pl
from jax.experimental.pallas import tpu as pltpu
from jax.experimental.pallas import tpu_sc as plsc

_SCALING = 2.0

_V = 1000000
_BATCH = 16384
_HIST = 50
_D = 64
_R = 16
_W = 128

_NC = 2
_NS = 16
_NW = _NC * _NS
_IROWS_PER_W = _BATCH // _NW
_CI = 4
_CHUNK = _CI * _HIST
_N_CHUNKS = _IROWS_PER_W // _CI

_L = 16
_NJ = _D // _L

_BCAST_DN = lax.GatherDimensionNumbers(
    offset_dims=(), collapsed_slice_dims=(0,), start_index_map=(0,))


def _bcast(vec, r):
    idx = jnp.full((_L, 1), r, jnp.int32)
    return lax.gather(vec, idx, _BCAST_DN, slice_sizes=(1,),
                      mode=lax.GatherScatterMode.PROMISE_IN_BOUNDS)


def _sc_fused(tab_hbm, idx_hbm, b_hbm, out_hbm,
              idx_v, g_v, out_v, b_v,
              sem_g0, sem_g1, sem_w0, sem_w1,
              sem_i0, sem_i1, sem_i2, sem_i3):
    sem_g = (sem_g0, sem_g1)
    sem_w = (sem_w0, sem_w1)
    sem_i = (sem_i0, sem_i1, sem_i2, sem_i3)
    wid = lax.axis_index("s") * _NC + lax.axis_index("c")
    irow_base = wid * _IROWS_PER_W

    pltpu.sync_copy(b_hbm, b_v)
    for r in range(_R):
        for j in range(_NJ):
            b_v[r, pl.ds(_L * j, _L)] = b_v[r, pl.ds(_L * j, _L)] * _SCALING

    def idx_desc(c, s):
        return (idx_hbm.at[pl.ds(irow_base + c * _CI, _CI)],
                idx_v.at[s], sem_i[s])

    def gather_descs(c, s, p):
        descs = []
        for j in range(_CI):
            descs.append((tab_hbm.at[idx_v.at[s, j, pl.ds(0, _HIST)]],
                          g_v.at[p].at[pl.ds(j * _HIST, _HIST)], sem_g[p]))
        return descs

    def issue(c, s, p):
        for src, dst, sem in gather_descs(c, s, p):
            pltpu.async_copy(src, dst, sem)

    def drain(c, s, p):
        for src, dst, sem in gather_descs(c, s, p):
            pltpu.make_async_copy(src, dst, sem).wait()

    def compute(p):
        b_regs = [[b_v[r, pl.ds(_L * j, _L)] for j in range(_NJ)]
                  for r in range(_R)]
        def qbody(q, carry):
            @plsc.parallel_loop(0, _HIST, unroll=5)
            def _row(rr):
                i = q * _HIST + rr
                a_row = g_v[p, i, pl.ds(_D, _R)]
                accs = [g_v[p, i, pl.ds(_L * j, _L)] for j in range(_NJ)]
                for r in range(_R):
                    ab = _bcast(a_row, r)
                    for j in range(_NJ):
                        accs[j] = accs[j] + ab * b_regs[r][j]
                for j in range(_NJ):
                    out_v[p, q, rr, pl.ds(_L * j, _L)] = accs[j]
            return carry

        lax.fori_loop(0, _CI, qbody, 0)

    def wb_desc(c, p):
        row0 = irow_base + c * _CI
        return (out_v.at[p], out_hbm.at[pl.ds(row0, _CI)], sem_w[p])

    for c in range(4):
        src, dst, sem = idx_desc(c, c)
        pltpu.async_copy(src, dst, sem)
    for c in range(2):
        src, dst, sem = idx_desc(c, c)
        pltpu.make_async_copy(src, dst, sem).wait()
        issue(c, c, c)

    def outer(k4, carry):
        for u in range(4):
            c = k4 * 4 + u
            p = u % 2
            drain(c, u, p)

            @pl.when(c >= 2)
            def _wait_prev_writeback():
                src, dst, sem = wb_desc(c - 2, p)
                pltpu.make_async_copy(src, dst, sem).wait()

            compute(p)
            src, dst, sem = wb_desc(c, p)
            pltpu.async_copy(src, dst, sem)

            @pl.when(c + 2 < _N_CHUNKS)
            def _prefetch_next():
                si, di, smi = idx_desc(c + 2, (u + 2) % 4)
                pltpu.make_async_copy(si, di, smi).wait()
                issue(c + 2, (u + 2) % 4, p)

            @pl.when(c + 4 < _N_CHUNKS)
            def _stage_next_idx():
                si, di, smi = idx_desc(c + 4, u)
                pltpu.async_copy(si, di, smi)
        return carry

    lax.fori_loop(0, _N_CHUNKS // 4, outer, 0)

    for p in range(2):
        src, dst, sem = wb_desc(_N_CHUNKS - 2 + p, p)
        pltpu.make_async_copy(src, dst, sem).wait()


_sc_fused_call = pl.kernel(
    _sc_fused,
    out_type=jax.ShapeDtypeStruct((_BATCH, _HIST, _D), jnp.float32),
    mesh=plsc.VectorSubcoreMesh(core_axis_name="c", subcore_axis_name="s"),
    compiler_params=pltpu.CompilerParams(use_tc_tiling_on_sc=True),
    scratch_types=[
        pltpu.VMEM((4, _CI, _W), jnp.int32),
        pltpu.VMEM((2, _CHUNK, _W), jnp.float32),
        pltpu.VMEM((2, _CI, _HIST, _D), jnp.float32),
        pltpu.VMEM((_R, _W), jnp.float32),
        pltpu.SemaphoreType.DMA,
        pltpu.SemaphoreType.DMA,
        pltpu.SemaphoreType.DMA,
        pltpu.SemaphoreType.DMA,
        pltpu.SemaphoreType.DMA,
        pltpu.SemaphoreType.DMA,
        pltpu.SemaphoreType.DMA,
        pltpu.SemaphoreType.DMA,
    ],
)


def kernel(inputs, embedding, lora_A, lora_B):
    packed = jnp.concatenate(
        [embedding, lora_A,
         jnp.zeros((_V, _W - _D - _R), jnp.float32)], axis=-1)
    idx_pad = jnp.pad(inputs.astype(jnp.int32),
                      ((0, 0), (0, _W - _HIST)))
    b_pad = jnp.pad(lora_B, ((0, 0), (0, _W - _D)))
    return _sc_fused_call(packed, idx_pad, b_pad)


# speedup vs baseline: 1.0660x; 1.0660x over previous
import jax
import jax.numpy as jnp
from jax import lax
from jax.experimental import pallas as pl
from jax.experimental.pallas import tpu as pltpu
from jax.experimental.pallas import tpu_sc as plsc

_SCALING = 2.0

_V = 1000000
_BATCH = 16384
_HIST = 50
_D = 64
_R = 16
_W = 128

_NC = 2
_NS = 16
_NW = _NC * _NS
_IROWS_PER_W = _BATCH // _NW
_CI = 4
_CHUNK = _CI * _HIST
_N_CHUNKS = _IROWS_PER_W // _CI

_L = 16
_NJ = _D // _L

_BCAST_DN = lax.GatherDimensionNumbers(
    offset_dims=(), collapsed_slice_dims=(0,), start_index_map=(0,))


def _bcast(vec, r):
    idx = jnp.full((_L, 1), r, jnp.int32)
    return lax.gather(vec, idx, _BCAST_DN, slice_sizes=(1,),
                      mode=lax.GatherScatterMode.PROMISE_IN_BOUNDS)


def _sc_fused(tab_hbm, idx_hbm, b_hbm, out_hbm,
              idx_v, g_v, out_v, b_v,
              sem_g0, sem_g1, sem_w0, sem_w1,
              sem_i0, sem_i1, sem_i2, sem_i3):
    sem_g = (sem_g0, sem_g1)
    sem_w = (sem_w0, sem_w1)
    sem_i = (sem_i0, sem_i1, sem_i2, sem_i3)
    wid = lax.axis_index("s") * _NC + lax.axis_index("c")
    irow_base = wid * _IROWS_PER_W

    pltpu.sync_copy(b_hbm, b_v)
    for r in range(_R):
        for j in range(_NJ):
            b_v[r, pl.ds(_L * j, _L)] = b_v[r, pl.ds(_L * j, _L)] * _SCALING

    def idx_desc(c, s):
        return (idx_hbm.at[pl.ds(irow_base + c * _CI, _CI)],
                idx_v.at[s], sem_i[s])

    def gather_descs(c, s, p):
        descs = []
        for j in range(_CI):
            descs.append((tab_hbm.at[idx_v.at[s, j, pl.ds(0, _HIST)]],
                          g_v.at[p].at[pl.ds(j * _HIST, _HIST)], sem_g[p]))
        return descs

    def issue(c, s, p):
        for src, dst, sem in gather_descs(c, s, p):
            pltpu.async_copy(src, dst, sem)

    def drain(c, s, p):
        for src, dst, sem in gather_descs(c, s, p):
            pltpu.make_async_copy(src, dst, sem).wait()

    def compute(p):
        b_regs = [[b_v[r, pl.ds(_L * j, _L)] for j in range(_NJ)]
                  for r in range(_R)]
        for q in range(_CI):
            @plsc.parallel_loop(0, _HIST, unroll=2)
            def _row(rr):
                i = q * _HIST + rr
                a_row = g_v[p, i, pl.ds(_D, _R)]
                accs = [g_v[p, i, pl.ds(_L * j, _L)] for j in range(_NJ)]
                for r in range(_R):
                    ab = _bcast(a_row, r)
                    for j in range(_NJ):
                        accs[j] = accs[j] + ab * b_regs[r][j]
                for j in range(_NJ):
                    out_v[p, q, rr, pl.ds(_L * j, _L)] = accs[j]

    def wb_desc(c, p):
        row0 = irow_base + c * _CI
        return (out_v.at[p], out_hbm.at[pl.ds(row0, _CI)], sem_w[p])

    for c in range(4):
        src, dst, sem = idx_desc(c, c)
        pltpu.async_copy(src, dst, sem)
    for c in range(2):
        src, dst, sem = idx_desc(c, c)
        pltpu.make_async_copy(src, dst, sem).wait()
        issue(c, c, c)

    def outer(k4, carry):
        for u in range(4):
            c = k4 * 4 + u
            p = u % 2
            drain(c, u, p)

            @pl.when(c >= 2)
            def _wait_prev_writeback():
                src, dst, sem = wb_desc(c - 2, p)
                pltpu.make_async_copy(src, dst, sem).wait()

            compute(p)
            src, dst, sem = wb_desc(c, p)
            pltpu.async_copy(src, dst, sem)

            @pl.when(c + 2 < _N_CHUNKS)
            def _prefetch_next():
                si, di, smi = idx_desc(c + 2, (u + 2) % 4)
                pltpu.make_async_copy(si, di, smi).wait()
                issue(c + 2, (u + 2) % 4, p)

            @pl.when(c + 4 < _N_CHUNKS)
            def _stage_next_idx():
                si, di, smi = idx_desc(c + 4, u)
                pltpu.async_copy(si, di, smi)
        return carry

    lax.fori_loop(0, _N_CHUNKS // 4, outer, 0)

    for p in range(2):
        src, dst, sem = wb_desc(_N_CHUNKS - 2 + p, p)
        pltpu.make_async_copy(src, dst, sem).wait()


_sc_fused_call = pl.kernel(
    _sc_fused,
    out_type=jax.ShapeDtypeStruct((_BATCH, _HIST, _D), jnp.float32),
    mesh=plsc.VectorSubcoreMesh(core_axis_name="c", subcore_axis_name="s"),
    compiler_params=pltpu.CompilerParams(use_tc_tiling_on_sc=True),
    scratch_types=[
        pltpu.VMEM((4, _CI, _W), jnp.int32),
        pltpu.VMEM((2, _CHUNK, _W), jnp.float32),
        pltpu.VMEM((2, _CI, _HIST, _D), jnp.float32),
        pltpu.VMEM((_R, _W), jnp.float32),
        pltpu.SemaphoreType.DMA,
        pltpu.SemaphoreType.DMA,
        pltpu.SemaphoreType.DMA,
        pltpu.SemaphoreType.DMA,
        pltpu.SemaphoreType.DMA,
        pltpu.SemaphoreType.DMA,
        pltpu.SemaphoreType.DMA,
        pltpu.SemaphoreType.DMA,
    ],
)


def kernel(inputs, embedding, lora_A, lora_B):
    packed = jnp.concatenate(
        [embedding, lora_A,
         jnp.zeros((_V, _W - _D - _R), jnp.float32)], axis=-1)
    idx_pad = jnp.pad(inputs.astype(jnp.int32),
                      ((0, 0), (0, _W - _HIST)))
    b_pad = jnp.pad(lora_B, ((0, 0), (0, _W - _D)))
    return _sc_fused_call(packed, idx_pad, b_pad)
